# baseline (speedup 1.0000x reference)
import jax
import jax.numpy as jnp
from jax import lax
from jax.experimental import pallas as pl
from jax.experimental.pallas import tpu as pltpu
from jax.experimental.pallas import tpu_sc as plsc

N = 100000
E = 1600000
F_IN = 37
H = 64
OUT = 3

LW = 128
CE = 256
ROWS_P = 12544
PAD = ROWS_P * LW - E
RPT = ROWS_P // 16
DROWS_SC = ROWS_P // 2
DRPT = DROWS_SC // 16
NPAD = 100352
STRIPE = NPAD // 16
ZR = 112
GC = 8
NG = RPT * LW // CE // GC
PREV_IB = {0: 2, 1: 0, 2: 1}
R = 1024
GRID = NPAD // R
BR = R // 8
PR = NPAD // LW
NSLICE = 4
SLW = H // NSLICE


def _deg_body(dstp_ref, p_ref, deg_s, didx, ones, zbuf, isem0, isem1):
    c = lax.axis_index("c")
    t = lax.axis_index("s")
    isems = (isem0, isem1)
    zeros16 = jnp.zeros((16,), jnp.float32)
    ones16 = jnp.ones((16,), jnp.float32)
    for i in range(CE // 16):
        ones[pl.ds(16 * i, 16)] = ones16

    def zb(i, _):
        zbuf[pl.ds(i * 16, 16)] = zeros16
        return 0

    lax.fori_loop(0, STRIPE // 16, zb, 0)
    sbase = pl.multiple_of(t * STRIPE, 128)
    pltpu.sync_copy(zbuf, deg_s.at[pl.ds(sbase, STRIPE)])
    plsc.subcore_barrier()

    base = pl.multiple_of((c * DROWS_SC + t * DRPT) * LW, 1024)
    nch = DRPT * LW // CE

    def idx_fire(k, b):
        off = pl.multiple_of(base + jnp.minimum(k, nch - 1) * CE, 256)
        pltpu.async_copy(dstp_ref.at[pl.ds(off, CE)], didx.at[b], isems[b])

    def idx_wait(b):
        pltpu.make_async_copy(dstp_ref.at[pl.ds(0, CE)], didx.at[b],
                              isems[b]).wait()

    idx_fire(0, 0)

    def ebody(j, _):
        idx_wait(0)
        idx_fire(2 * j + 1, 1)
        pltpu.sync_copy(ones, deg_s.at[didx.at[0]], add=True)
        idx_wait(1)
        idx_fire(2 * j + 2, 0)
        pltpu.sync_copy(ones, deg_s.at[didx.at[1]], add=True)
        return 0

    lax.fori_loop(0, nch // 2, ebody, 0)
    idx_wait(0)
    plsc.subcore_barrier()
    obase = pl.multiple_of(c * NPAD + t * STRIPE, 128)
    pltpu.sync_copy(deg_s.at[pl.ds(sbase, STRIPE)],
                    p_ref.at[pl.ds(obase, STRIPE)])


def _agg_body(gf_ref, src2_ref, dst2_ref, acc_ref, acc_s, sidx, didx, rows,
              zbuf, gsem0, gsem1, gsem2, gsem3, ssem0, ssem1, ssem2, ssem3,
              isem0, isem1, isem2):
    c = lax.axis_index("c")
    t = lax.axis_index("s")
    gsems = (gsem0, gsem1, gsem2, gsem3)
    ssems = (ssem0, ssem1, ssem2, ssem3)
    isems = (isem0, isem1, isem2)
    zeros16 = jnp.zeros((16,), jnp.float32)

    def zb(i, _):
        zbuf[i, :] = zeros16
        return 0

    lax.fori_loop(0, ZR, zb, 0)
    stripe = pl.multiple_of(t * STRIPE, 16)
    gbase = pl.multiple_of(t * (RPT * LW // CE), 8)

    def idx_fire(g, ib):
        row = pl.multiple_of(gbase + g * GC, 8)
        pltpu.async_copy(src2_ref.at[pl.ds(row, GC)], sidx.at[ib], isems[ib])
        pltpu.async_copy(dst2_ref.at[pl.ds(row, GC)], didx.at[ib], isems[ib])

    def idx_wait(ib):
        pltpu.make_async_copy(src2_ref.at[pl.ds(0, GC)], sidx.at[ib],
                              isems[ib]).wait()
        pltpu.make_async_copy(dst2_ref.at[pl.ds(0, GC)], didx.at[ib],
                              isems[ib]).wait()

    for s_loc in range(2):
        sid = 2 * s_loc + c
        for k in range(STRIPE // ZR):
            pltpu.sync_copy(zbuf, acc_s.at[pl.ds(stripe + ZR * k, ZR)])
        plsc.subcore_barrier()

        offv = jnp.full((16,), sid * NPAD, jnp.int32)

        def fire_gather(ib, r, b):
            for kk in range(CE // 16):
                sidx[ib, r, pl.ds(16 * kk, 16)] = (
                    sidx[ib, r, pl.ds(16 * kk, 16)] + offv)
            pltpu.async_copy(gf_ref.at[sidx.at[ib, r]], rows.at[b], gsems[b])

        def gather_wait(b):
            pltpu.make_async_copy(gf_ref.at[sidx.at[0, 0]], rows.at[b],
                                  gsems[b]).wait()

        def scatter_fire(ib, r, b):
            pltpu.async_copy(rows.at[b], acc_s.at[didx.at[ib, r]], ssems[b],
                             add=True)

        def scatter_wait(b):
            pltpu.make_async_copy(rows.at[b], acc_s.at[didx.at[0, 0]],
                                  ssems[b]).wait()

        def do_group(ib, first):
            for r in range(GC):
                b = r % 4
                bp = (b + 2) % 4
                if first and r < 4:
                    pass
                else:
                    scatter_wait(b)
                fire_gather(ib, r, b)
                if not (first and r < 2):
                    gather_wait(bp)
                    if r >= 2:
                        scatter_fire(ib, r - 2, bp)
                    else:
                        scatter_fire(PREV_IB[ib], GC - 2 + r, bp)

        pltpu.sync_copy(src2_ref.at[pl.ds(gbase, GC)], sidx.at[0])
        pltpu.sync_copy(dst2_ref.at[pl.ds(gbase, GC)], didx.at[0])
        idx_fire(1, 1)
        do_group(0, True)

        def gbody(jj, _):
            for p in range(3):
                g = 3 * jj + 1 + p
                ib = (1 + p) % 3
                nxt = (ib + 1) % 3
                idx_fire(jnp.minimum(g + 1, NG - 1), nxt)
                idx_wait(ib)
                do_group(ib, False)
            return 0

        lax.fori_loop(0, (NG - 1) // 3, gbody, 0)

        gather_wait(2)
        scatter_fire(0, GC - 2, 2)
        gather_wait(3)
        scatter_fire(0, GC - 1, 3)
        for b in range(4):
            scatter_wait(b)
        idx_wait(1)
        plsc.subcore_barrier()

        obase = pl.multiple_of(sid * NPAD + t * STRIPE, 16)
        for k in range(STRIPE // ZR):
            pltpu.sync_copy(acc_s.at[pl.ds(stripe + ZR * k, ZR)],
                            acc_ref.at[pl.ds(obase + ZR * k, ZR)])
        plsc.subcore_barrier()


def _dinv_packed(p_ref):
    dinv = lax.rsqrt(p_ref[0] + p_ref[1] + 1.0)
    dinv_t = dinv.T
    return jnp.concatenate(
        [jnp.broadcast_to(dinv_t[:, a:a + 1], (BR, SLW)) for a in range(8)],
        axis=1)


def _pack(v):
    return jnp.concatenate([v[BR * a:BR * (a + 1), :] for a in range(8)],
                           axis=1)


def _unpack(vp):
    return jnp.concatenate([vp[:, SLW * a:SLW * (a + 1)] for a in range(8)],
                           axis=0)


def _stage_a_body(x_ref, w1_ref, g_ref):
    hw = jnp.dot(x_ref[...], w1_ref[...],
                 preferred_element_type=jnp.float32)
    for s in range(NSLICE):
        g_ref[s] = _pack(hw[:, SLW * s:SLW * (s + 1)])


def _stage_b_body(p_ref, acc_ref, g_ref, b1p_ref, m2_ref, g2_ref):
    dinv_p = _dinv_packed(p_ref)
    hs_p = [jnp.maximum((acc_ref[s] + g_ref[s]) * dinv_p + b1p_ref[s], 0.0)
            for s in range(NSLICE)]
    for sp in range(NSLICE):
        hw2 = jnp.zeros((BR, LW), jnp.float32)
        for s in range(NSLICE):
            hw2 = hw2 + jnp.dot(hs_p[s], m2_ref[s, sp],
                                preferred_element_type=jnp.float32)
        g2_ref[sp] = hw2 * dinv_p


def _stage_c_body(p_ref, acc_ref, g_ref, b2p_ref, ml_ref, bl_ref, o_ref):
    dinv_p = _dinv_packed(p_ref)
    logits_p = jnp.zeros((BR, 8 * OUT), jnp.float32)
    for s in range(NSLICE):
        hs_p = jnp.maximum(
            (acc_ref[s] + g_ref[s]) * dinv_p + b2p_ref[s], 0.0)
        logits_p = logits_p + jnp.dot(hs_p, ml_ref[s],
                                      preferred_element_type=jnp.float32)
    logits = jnp.concatenate(
        [logits_p[:, OUT * a:OUT * (a + 1)] for a in range(8)],
        axis=0) + bl_ref[...]
    m = jnp.max(logits, axis=1, keepdims=True)
    e = jnp.exp(logits - m)
    o_ref[...] = e / jnp.sum(e, axis=1, keepdims=True)


_SC_MESH = plsc.VectorSubcoreMesh(core_axis_name="c", subcore_axis_name="s")
_SC_PARAMS = pltpu.CompilerParams(use_tc_tiling_on_sc=False)

_deg_kernel = pl.kernel(
    _deg_body,
    out_type=jax.ShapeDtypeStruct((2 * NPAD,), jnp.float32),
    mesh=_SC_MESH,
    compiler_params=_SC_PARAMS,
    scratch_types=[
        pltpu.MemorySpace.VMEM_SHARED((NPAD,), jnp.float32),
        pltpu.VMEM((2, CE), jnp.int32),
        pltpu.VMEM((CE,), jnp.float32),
        pltpu.VMEM((STRIPE,), jnp.float32),
        pltpu.SemaphoreType.DMA,
        pltpu.SemaphoreType.DMA,
    ],
)

_agg_kernel = pl.kernel(
    _agg_body,
    out_type=jax.ShapeDtypeStruct((NSLICE * NPAD, SLW), jnp.float32),
    mesh=_SC_MESH,
    compiler_params=_SC_PARAMS,
    scratch_types=[
        pltpu.MemorySpace.VMEM_SHARED((NPAD, SLW), jnp.float32),
        pltpu.VMEM((3, GC, CE), jnp.int32),
        pltpu.VMEM((3, GC, CE), jnp.int32),
        pltpu.VMEM((4, CE, SLW), jnp.float32),
        pltpu.VMEM((ZR, SLW), jnp.float32),
        pltpu.SemaphoreType.DMA,
        pltpu.SemaphoreType.DMA,
        pltpu.SemaphoreType.DMA,
        pltpu.SemaphoreType.DMA,
        pltpu.SemaphoreType.DMA,
        pltpu.SemaphoreType.DMA,
        pltpu.SemaphoreType.DMA,
        pltpu.SemaphoreType.DMA,
        pltpu.SemaphoreType.DMA,
        pltpu.SemaphoreType.DMA,
        pltpu.SemaphoreType.DMA,
    ],
)

_stage_a = pl.pallas_call(
    _stage_a_body,
    grid=(GRID,),
    in_specs=[
        pl.BlockSpec((R, F_IN), lambda i: (i, 0)),
        pl.BlockSpec((F_IN, H), lambda i: (0, 0)),
    ],
    out_specs=pl.BlockSpec((NSLICE, BR, LW), lambda i: (0, i, 0)),
    out_shape=jax.ShapeDtypeStruct((NSLICE, NPAD // 8, LW), jnp.float32),
)

_stage_b = pl.pallas_call(
    _stage_b_body,
    grid=(GRID,),
    in_specs=[
        pl.BlockSpec((2, 8, LW), lambda i: (0, i, 0)),
        pl.BlockSpec((NSLICE, BR, LW), lambda i: (0, i, 0)),
        pl.BlockSpec((NSLICE, BR, LW), lambda i: (0, i, 0)),
        pl.BlockSpec((NSLICE, LW), lambda i: (0, 0)),
        pl.BlockSpec((NSLICE, NSLICE, LW, LW), lambda i: (0, 0, 0, 0)),
    ],
    out_specs=pl.BlockSpec((NSLICE, BR, LW), lambda i: (0, i, 0)),
    out_shape=jax.ShapeDtypeStruct((NSLICE, NPAD // 8, LW), jnp.float32),
)

_stage_c = pl.pallas_call(
    _stage_c_body,
    grid=(GRID,),
    in_specs=[
        pl.BlockSpec((2, 8, LW), lambda i: (0, i, 0)),
        pl.BlockSpec((NSLICE, BR, LW), lambda i: (0, i, 0)),
        pl.BlockSpec((NSLICE, BR, LW), lambda i: (0, i, 0)),
        pl.BlockSpec((NSLICE, LW), lambda i: (0, 0)),
        pl.BlockSpec((NSLICE, LW, 8 * OUT), lambda i: (0, 0, 0)),
        pl.BlockSpec((1, OUT), lambda i: (0, 0)),
    ],
    out_specs=pl.BlockSpec((R, OUT), lambda i: (i, 0)),
    out_shape=jax.ShapeDtypeStruct((N, OUT), jnp.float32),
)


def _pack_bias(b):
    return jnp.broadcast_to(b.reshape(NSLICE, 1, SLW),
                            (NSLICE, 8, SLW)).reshape(NSLICE, LW)


def _block_diag_w(w):
    eye8 = jnp.eye(8, dtype=w.dtype)
    return jnp.stack([jnp.kron(eye8, w[SLW * s:SLW * (s + 1), :])
                      for s in range(NSLICE)])


def _block_diag_w2(w2):
    eye8 = jnp.eye(8, dtype=w2.dtype)
    return jnp.stack([
        jnp.stack([jnp.kron(eye8,
                            w2[SLW * s:SLW * (s + 1), SLW * sp:SLW * (sp + 1)])
                   for sp in range(NSLICE)])
        for s in range(NSLICE)])


def _perm(n):
    return (n & ~1023) | ((n & 127) << 3) | ((n >> 7) & 7)


def kernel(x, edge_index, W1, b1, W2, b2, Wl, bl):
    edge_index = edge_index.astype(jnp.int32)
    src = edge_index[0]
    dst = edge_index[1]
    pad_src = jnp.arange(PAD, dtype=jnp.int32) % N
    pad_dst = N + (jnp.arange(PAD, dtype=jnp.int32) % 8)
    srcf = jnp.concatenate([src, pad_src])
    dstf = jnp.concatenate([dst, pad_dst])
    src2 = _perm(srcf).reshape(-1, CE)
    dst2 = _perm(dstf).reshape(-1, CE)

    p = _deg_kernel(dstf).reshape(2, PR, LW)

    dinv = lax.rsqrt(p[0] + p[1] + 1.0)
    dinv_p = jnp.broadcast_to(
        dinv.reshape(GRID, 8, BR).transpose(0, 2, 1)[..., None],
        (GRID, BR, 8, SLW)).reshape(NPAD // 8, LW)

    g1 = _stage_a(x, W1) * dinv_p
    acc1 = _agg_kernel(g1.reshape(NSLICE * NPAD, SLW), src2, dst2)
    g2 = _stage_b(p, acc1.reshape(NSLICE, NPAD // 8, LW), g1,
                  _pack_bias(b1), _block_diag_w2(W2))
    acc2 = _agg_kernel(g2.reshape(NSLICE * NPAD, SLW), src2, dst2)
    return _stage_c(p, acc2.reshape(NSLICE, NPAD // 8, LW), g2,
                    _pack_bias(b2), _block_diag_w(Wl), bl.reshape(1, OUT))


# speedup vs baseline: 1.0400x; 1.0400x over previous
import jax
import jax.numpy as jnp
from jax import lax
from jax.experimental import pallas as pl
from jax.experimental.pallas import tpu as pltpu
from jax.experimental.pallas import tpu_sc as plsc

N = 100000
E = 1600000
F_IN = 37
H = 64
OUT = 3

LW = 128
CE = 256
ROWS_P = 12544
PAD = ROWS_P * LW - E
RPT = ROWS_P // 16
DROWS_SC = ROWS_P // 2
DRPT = DROWS_SC // 16
NPAD = 100352
STRIPE = NPAD // 16
ZR = 112
GC = 8
NG = RPT * LW // CE // GC
PREV_IB = {0: 2, 1: 0, 2: 1}
R = 1024
GRID = NPAD // R
BR = R // 8
PR = NPAD // LW
NSLICE = 4
SLW = H // NSLICE


def _deg_body(dstp_ref, p_ref, deg_s, didx, ones, zbuf, isem0, isem1):
    c = lax.axis_index("c")
    t = lax.axis_index("s")
    isems = (isem0, isem1)
    zeros16 = jnp.zeros((16,), jnp.float32)
    ones16 = jnp.ones((16,), jnp.float32)
    for i in range(CE // 16):
        ones[pl.ds(16 * i, 16)] = ones16

    def zb(i, _):
        zbuf[pl.ds(i * 16, 16)] = zeros16
        return 0

    lax.fori_loop(0, STRIPE // 16, zb, 0)
    sbase = pl.multiple_of(t * STRIPE, 128)
    pltpu.sync_copy(zbuf, deg_s.at[pl.ds(sbase, STRIPE)])
    plsc.subcore_barrier()

    base = pl.multiple_of((c * DROWS_SC + t * DRPT) * LW, 1024)
    nch = DRPT * LW // CE

    def idx_fire(k, b):
        off = pl.multiple_of(base + jnp.minimum(k, nch - 1) * CE, 256)
        pltpu.async_copy(dstp_ref.at[pl.ds(off, CE)], didx.at[b], isems[b])

    def idx_wait(b):
        pltpu.make_async_copy(dstp_ref.at[pl.ds(0, CE)], didx.at[b],
                              isems[b]).wait()

    idx_fire(0, 0)

    def ebody(j, _):
        idx_wait(0)
        idx_fire(2 * j + 1, 1)
        pltpu.sync_copy(ones, deg_s.at[didx.at[0]], add=True)
        idx_wait(1)
        idx_fire(2 * j + 2, 0)
        pltpu.sync_copy(ones, deg_s.at[didx.at[1]], add=True)
        return 0

    lax.fori_loop(0, nch // 2, ebody, 0)
    idx_wait(0)
    plsc.subcore_barrier()
    obase = pl.multiple_of(c * NPAD + t * STRIPE, 128)
    pltpu.sync_copy(deg_s.at[pl.ds(sbase, STRIPE)],
                    p_ref.at[pl.ds(obase, STRIPE)])


def _agg_body(gf_ref, src2_ref, dst2_ref, acc_ref, acc_s, sidx, didx, rows,
              zbuf, gsem0, gsem1, gsem2, gsem3, ssem0, ssem1, ssem2, ssem3,
              isem0, isem1, isem2):
    c = lax.axis_index("c")
    t = lax.axis_index("s")
    gsems = (gsem0, gsem1, gsem2, gsem3)
    ssems = (ssem0, ssem1, ssem2, ssem3)
    isems = (isem0, isem1, isem2)
    zeros16 = jnp.zeros((16,), jnp.float32)

    def zb(i, _):
        zbuf[i, :] = zeros16
        return 0

    lax.fori_loop(0, ZR, zb, 0)
    stripe = pl.multiple_of(t * STRIPE, 16)
    gbase = pl.multiple_of(t * (RPT * LW // CE), 8)

    def idx_fire(g, ib):
        row = pl.multiple_of(gbase + g * GC, 8)
        pltpu.async_copy(src2_ref.at[pl.ds(row, GC)], sidx.at[ib], isems[ib])
        pltpu.async_copy(dst2_ref.at[pl.ds(row, GC)], didx.at[ib], isems[ib])

    def idx_wait(ib):
        pltpu.make_async_copy(src2_ref.at[pl.ds(0, GC)], sidx.at[ib],
                              isems[ib]).wait()
        pltpu.make_async_copy(dst2_ref.at[pl.ds(0, GC)], didx.at[ib],
                              isems[ib]).wait()

    for s_loc in range(2):
        sid = 2 * s_loc + c
        for k in range(STRIPE // ZR):
            pltpu.sync_copy(zbuf, acc_s.at[pl.ds(stripe + ZR * k, ZR)])
        plsc.subcore_barrier()

        offv = jnp.full((16,), sid * NPAD, jnp.int32)

        def fire_gather(ib, r, b):
            for kk in range(CE // 16):
                sidx[ib, r, pl.ds(16 * kk, 16)] = (
                    sidx[ib, r, pl.ds(16 * kk, 16)] + offv)
            pltpu.async_copy(gf_ref.at[sidx.at[ib, r]], rows.at[b], gsems[b])

        def gather_wait(b):
            pltpu.make_async_copy(gf_ref.at[sidx.at[0, 0]], rows.at[b],
                                  gsems[b]).wait()

        def scatter_fire(ib, r, b):
            pltpu.async_copy(rows.at[b], acc_s.at[didx.at[ib, r]], ssems[b],
                             add=True)

        def scatter_wait(b):
            pltpu.make_async_copy(rows.at[b], acc_s.at[didx.at[0, 0]],
                                  ssems[b]).wait()

        def do_group(ib, first):
            for r in range(GC):
                b = r % 4
                bp = (b + 2) % 4
                if first and r < 4:
                    pass
                else:
                    scatter_wait(b)
                fire_gather(ib, r, b)
                if not (first and r < 2):
                    gather_wait(bp)
                    if r >= 2:
                        scatter_fire(ib, r - 2, bp)
                    else:
                        scatter_fire(PREV_IB[ib], GC - 2 + r, bp)

        pltpu.sync_copy(src2_ref.at[pl.ds(gbase, GC)], sidx.at[0])
        pltpu.sync_copy(dst2_ref.at[pl.ds(gbase, GC)], didx.at[0])
        idx_fire(1, 1)
        do_group(0, True)

        def gbody(jj, _):
            for p in range(3):
                g = 3 * jj + 1 + p
                ib = (1 + p) % 3
                nxt = (ib + 1) % 3
                idx_fire(jnp.minimum(g + 1, NG - 1), nxt)
                idx_wait(ib)
                do_group(ib, False)
            return 0

        lax.fori_loop(0, (NG - 1) // 3, gbody, 0)

        gather_wait(2)
        scatter_fire(0, GC - 2, 2)
        gather_wait(3)
        scatter_fire(0, GC - 1, 3)
        for b in range(4):
            scatter_wait(b)
        idx_wait(1)
        plsc.subcore_barrier()

        obase = pl.multiple_of(sid * NPAD + t * STRIPE, 16)
        for k in range(STRIPE // ZR):
            pltpu.sync_copy(acc_s.at[pl.ds(stripe + ZR * k, ZR)],
                            acc_ref.at[pl.ds(obase + ZR * k, ZR)])
        plsc.subcore_barrier()


def _dinv_packed(p_ref):
    dinv = lax.rsqrt(p_ref[0] + p_ref[1] + 1.0)
    dinv_t = dinv.T
    return jnp.concatenate(
        [jnp.broadcast_to(dinv_t[:, a:a + 1], (BR, SLW)) for a in range(8)],
        axis=1)


def _pack(v):
    return jnp.concatenate([v[BR * a:BR * (a + 1), :] for a in range(8)],
                           axis=1)


def _unpack(vp):
    return jnp.concatenate([vp[:, SLW * a:SLW * (a + 1)] for a in range(8)],
                           axis=0)


def _stage_a_body(p_ref, x_ref, w1_ref, g_ref):
    dinv_p = _dinv_packed(p_ref)
    hw = jnp.dot(x_ref[...], w1_ref[...],
                 preferred_element_type=jnp.float32)
    for s in range(NSLICE):
        g_ref[s] = _pack(hw[:, SLW * s:SLW * (s + 1)]) * dinv_p


def _stage_b_body(p_ref, acc_ref, g_ref, b1p_ref, m2_ref, g2_ref):
    dinv_p = _dinv_packed(p_ref)
    hs_p = [jnp.maximum((acc_ref[s] + g_ref[s]) * dinv_p + b1p_ref[s], 0.0)
            for s in range(NSLICE)]
    for sp in range(NSLICE):
        hw2 = jnp.zeros((BR, LW), jnp.float32)
        for s in range(NSLICE):
            hw2 = hw2 + jnp.dot(hs_p[s], m2_ref[s, sp],
                                preferred_element_type=jnp.float32)
        g2_ref[sp] = hw2 * dinv_p


def _stage_c_body(p_ref, acc_ref, g_ref, b2p_ref, ml_ref, bl_ref, o_ref):
    dinv_p = _dinv_packed(p_ref)
    logits_p = jnp.zeros((BR, 8 * OUT), jnp.float32)
    for s in range(NSLICE):
        hs_p = jnp.maximum(
            (acc_ref[s] + g_ref[s]) * dinv_p + b2p_ref[s], 0.0)
        logits_p = logits_p + jnp.dot(hs_p, ml_ref[s],
                                      preferred_element_type=jnp.float32)
    logits = jnp.concatenate(
        [logits_p[:, OUT * a:OUT * (a + 1)] for a in range(8)],
        axis=0) + bl_ref[...]
    m = jnp.max(logits, axis=1, keepdims=True)
    e = jnp.exp(logits - m)
    o_ref[...] = e / jnp.sum(e, axis=1, keepdims=True)


_SC_MESH = plsc.VectorSubcoreMesh(core_axis_name="c", subcore_axis_name="s")
_SC_PARAMS = pltpu.CompilerParams(use_tc_tiling_on_sc=False)

_deg_kernel = pl.kernel(
    _deg_body,
    out_type=jax.ShapeDtypeStruct((2 * NPAD,), jnp.float32),
    mesh=_SC_MESH,
    compiler_params=_SC_PARAMS,
    scratch_types=[
        pltpu.MemorySpace.VMEM_SHARED((NPAD,), jnp.float32),
        pltpu.VMEM((2, CE), jnp.int32),
        pltpu.VMEM((CE,), jnp.float32),
        pltpu.VMEM((STRIPE,), jnp.float32),
        pltpu.SemaphoreType.DMA,
        pltpu.SemaphoreType.DMA,
    ],
)

_agg_kernel = pl.kernel(
    _agg_body,
    out_type=jax.ShapeDtypeStruct((NSLICE * NPAD, SLW), jnp.float32),
    mesh=_SC_MESH,
    compiler_params=_SC_PARAMS,
    scratch_types=[
        pltpu.MemorySpace.VMEM_SHARED((NPAD, SLW), jnp.float32),
        pltpu.VMEM((3, GC, CE), jnp.int32),
        pltpu.VMEM((3, GC, CE), jnp.int32),
        pltpu.VMEM((4, CE, SLW), jnp.float32),
        pltpu.VMEM((ZR, SLW), jnp.float32),
        pltpu.SemaphoreType.DMA,
        pltpu.SemaphoreType.DMA,
        pltpu.SemaphoreType.DMA,
        pltpu.SemaphoreType.DMA,
        pltpu.SemaphoreType.DMA,
        pltpu.SemaphoreType.DMA,
        pltpu.SemaphoreType.DMA,
        pltpu.SemaphoreType.DMA,
        pltpu.SemaphoreType.DMA,
        pltpu.SemaphoreType.DMA,
        pltpu.SemaphoreType.DMA,
    ],
)

_stage_a = pl.pallas_call(
    _stage_a_body,
    grid=(GRID,),
    in_specs=[
        pl.BlockSpec((2, 8, LW), lambda i: (0, i, 0)),
        pl.BlockSpec((R, F_IN), lambda i: (i, 0)),
        pl.BlockSpec((F_IN, H), lambda i: (0, 0)),
    ],
    out_specs=pl.BlockSpec((NSLICE, BR, LW), lambda i: (0, i, 0)),
    out_shape=jax.ShapeDtypeStruct((NSLICE, NPAD // 8, LW), jnp.float32),
)

_stage_b = pl.pallas_call(
    _stage_b_body,
    grid=(GRID,),
    in_specs=[
        pl.BlockSpec((2, 8, LW), lambda i: (0, i, 0)),
        pl.BlockSpec((NSLICE, BR, LW), lambda i: (0, i, 0)),
        pl.BlockSpec((NSLICE, BR, LW), lambda i: (0, i, 0)),
        pl.BlockSpec((NSLICE, LW), lambda i: (0, 0)),
        pl.BlockSpec((NSLICE, NSLICE, LW, LW), lambda i: (0, 0, 0, 0)),
    ],
    out_specs=pl.BlockSpec((NSLICE, BR, LW), lambda i: (0, i, 0)),
    out_shape=jax.ShapeDtypeStruct((NSLICE, NPAD // 8, LW), jnp.float32),
)

_stage_c = pl.pallas_call(
    _stage_c_body,
    grid=(GRID,),
    in_specs=[
        pl.BlockSpec((2, 8, LW), lambda i: (0, i, 0)),
        pl.BlockSpec((NSLICE, BR, LW), lambda i: (0, i, 0)),
        pl.BlockSpec((NSLICE, BR, LW), lambda i: (0, i, 0)),
        pl.BlockSpec((NSLICE, LW), lambda i: (0, 0)),
        pl.BlockSpec((NSLICE, LW, 8 * OUT), lambda i: (0, 0, 0)),
        pl.BlockSpec((1, OUT), lambda i: (0, 0)),
    ],
    out_specs=pl.BlockSpec((R, OUT), lambda i: (i, 0)),
    out_shape=jax.ShapeDtypeStruct((N, OUT), jnp.float32),
)


def _pack_bias(b):
    return jnp.broadcast_to(b.reshape(NSLICE, 1, SLW),
                            (NSLICE, 8, SLW)).reshape(NSLICE, LW)


def _block_diag_w(w):
    eye8 = jnp.eye(8, dtype=w.dtype)
    return jnp.stack([jnp.kron(eye8, w[SLW * s:SLW * (s + 1), :])
                      for s in range(NSLICE)])


def _block_diag_w2(w2):
    eye8 = jnp.eye(8, dtype=w2.dtype)
    return jnp.stack([
        jnp.stack([jnp.kron(eye8,
                            w2[SLW * s:SLW * (s + 1), SLW * sp:SLW * (sp + 1)])
                   for sp in range(NSLICE)])
        for s in range(NSLICE)])


def _perm(n):
    return (n & ~1023) | ((n & 127) << 3) | ((n >> 7) & 7)


def kernel(x, edge_index, W1, b1, W2, b2, Wl, bl):
    edge_index = edge_index.astype(jnp.int32)
    src = edge_index[0]
    dst = edge_index[1]
    pad_src = jnp.arange(PAD, dtype=jnp.int32) % N
    pad_dst = N + (jnp.arange(PAD, dtype=jnp.int32) % 8)
    srcf = jnp.concatenate([src, pad_src])
    dstf = jnp.concatenate([dst, pad_dst])
    src2 = _perm(srcf).reshape(-1, CE)
    dst2 = _perm(dstf).reshape(-1, CE)

    p = _deg_kernel(dstf).reshape(2, PR, LW)

    g1 = _stage_a(p, x, W1)
    acc1 = _agg_kernel(g1.reshape(NSLICE * NPAD, SLW), src2, dst2)
    g2 = _stage_b(p, acc1.reshape(NSLICE, NPAD // 8, LW), g1,
                  _pack_bias(b1), _block_diag_w2(W2))
    acc2 = _agg_kernel(g2.reshape(NSLICE * NPAD, SLW), src2, dst2)
    return _stage_c(p, acc2.reshape(NSLICE, NPAD // 8, LW), g2,
                    _pack_bias(b2), _block_diag_w(Wl), bl.reshape(1, OUT))


# speedup vs baseline: 1.0669x; 1.0258x over previous
import jax
import jax.numpy as jnp
from jax import lax
from jax.experimental import pallas as pl
from jax.experimental.pallas import tpu as pltpu
from jax.experimental.pallas import tpu_sc as plsc

N = 100000
E = 1600000
F_IN = 37
H = 64
OUT = 3

LW = 128
CE = 256
ROWS_P = 12544
PAD = ROWS_P * LW - E
RPT = ROWS_P // 16
DROWS_SC = ROWS_P // 2
DRPT = DROWS_SC // 16
NPAD = 100352
STRIPE = NPAD // 16
ZR = 112
GC = 8
NG = RPT * LW // CE // GC
PREV_IB = {0: 2, 1: 0, 2: 1}
R = 1024
GRID = NPAD // R
BR = R // 8
PR = NPAD // LW
NSLICE = 4
SLW = H // NSLICE


def _deg_body(dstp_ref, p_ref, deg_s, didx, ones, zbuf, isem0, isem1):
    c = lax.axis_index("c")
    t = lax.axis_index("s")
    isems = (isem0, isem1)
    zeros16 = jnp.zeros((16,), jnp.float32)
    ones16 = jnp.ones((16,), jnp.float32)
    for i in range(CE // 16):
        ones[pl.ds(16 * i, 16)] = ones16

    def zb(i, _):
        zbuf[pl.ds(i * 16, 16)] = zeros16
        return 0

    lax.fori_loop(0, STRIPE // 16, zb, 0)
    sbase = pl.multiple_of(t * STRIPE, 128)
    pltpu.sync_copy(zbuf, deg_s.at[pl.ds(sbase, STRIPE)])
    plsc.subcore_barrier()

    base = pl.multiple_of((c * DROWS_SC + t * DRPT) * LW, 1024)
    nch = DRPT * LW // CE

    def idx_fire(k, b):
        off = pl.multiple_of(base + jnp.minimum(k, nch - 1) * CE, 256)
        pltpu.async_copy(dstp_ref.at[pl.ds(off, CE)], didx.at[b], isems[b])

    def idx_wait(b):
        pltpu.make_async_copy(dstp_ref.at[pl.ds(0, CE)], didx.at[b],
                              isems[b]).wait()

    idx_fire(0, 0)

    def ebody(j, _):
        idx_wait(0)
        idx_fire(2 * j + 1, 1)
        pltpu.sync_copy(ones, deg_s.at[didx.at[0]], add=True)
        idx_wait(1)
        idx_fire(2 * j + 2, 0)
        pltpu.sync_copy(ones, deg_s.at[didx.at[1]], add=True)
        return 0

    lax.fori_loop(0, nch // 2, ebody, 0)
    idx_wait(0)
    plsc.subcore_barrier()
    obase = pl.multiple_of(c * NPAD + t * STRIPE, 128)
    pltpu.sync_copy(deg_s.at[pl.ds(sbase, STRIPE)],
                    p_ref.at[pl.ds(obase, STRIPE)])


def _agg_body(gf_ref, src2_ref, dst2_ref, acc_ref, acc_s, sidx, didx, rows,
              zbuf, gsem0, gsem1, gsem2, gsem3, ssem0, ssem1, ssem2, ssem3,
              isem0, isem1, isem2):
    c = lax.axis_index("c")
    t = lax.axis_index("s")
    gsems = (gsem0, gsem1, gsem2, gsem3)
    ssems = (ssem0, ssem1, ssem2, ssem3)
    isems = (isem0, isem1, isem2)
    zeros16 = jnp.zeros((16,), jnp.float32)

    def zb(i, _):
        zbuf[i, :] = zeros16
        return 0

    lax.fori_loop(0, ZR, zb, 0)
    stripe = pl.multiple_of(t * STRIPE, 16)
    gbase = pl.multiple_of(t * (RPT * LW // CE), 8)

    def idx_fire(g, ib):
        row = pl.multiple_of(gbase + g * GC, 8)
        pltpu.async_copy(src2_ref.at[pl.ds(row, GC)], sidx.at[ib], isems[ib])
        pltpu.async_copy(dst2_ref.at[pl.ds(row, GC)], didx.at[ib], isems[ib])

    def idx_wait(ib):
        pltpu.make_async_copy(src2_ref.at[pl.ds(0, GC)], sidx.at[ib],
                              isems[ib]).wait()
        pltpu.make_async_copy(dst2_ref.at[pl.ds(0, GC)], didx.at[ib],
                              isems[ib]).wait()

    for s_loc in range(2):
        sid = 2 * s_loc + c
        for k in range(STRIPE // ZR):
            pltpu.sync_copy(zbuf, acc_s.at[pl.ds(stripe + ZR * k, ZR)])
        plsc.subcore_barrier()

        offv = jnp.full((16,), sid * NPAD, jnp.int32)

        def fire_gather(ib, r, b):
            for kk in range(CE // 16):
                sidx[ib, r, pl.ds(16 * kk, 16)] = (
                    sidx[ib, r, pl.ds(16 * kk, 16)] + offv)
            pltpu.async_copy(gf_ref.at[sidx.at[ib, r]], rows.at[b], gsems[b])

        def gather_wait(b):
            pltpu.make_async_copy(gf_ref.at[sidx.at[0, 0]], rows.at[b],
                                  gsems[b]).wait()

        def scatter_fire(ib, r, b):
            pltpu.async_copy(rows.at[b], acc_s.at[didx.at[ib, r]], ssems[b],
                             add=True)

        def scatter_wait(b):
            pltpu.make_async_copy(rows.at[b], acc_s.at[didx.at[0, 0]],
                                  ssems[b]).wait()

        def do_group(ib, first):
            for r in range(GC):
                b = r % 4
                bp = (b + 2) % 4
                if first and r < 4:
                    pass
                else:
                    scatter_wait(b)
                fire_gather(ib, r, b)
                if not (first and r < 2):
                    gather_wait(bp)
                    if r >= 2:
                        scatter_fire(ib, r - 2, bp)
                    else:
                        scatter_fire(PREV_IB[ib], GC - 2 + r, bp)

        pltpu.sync_copy(src2_ref.at[pl.ds(gbase, GC)], sidx.at[0])
        pltpu.sync_copy(dst2_ref.at[pl.ds(gbase, GC)], didx.at[0])
        idx_fire(1, 1)
        do_group(0, True)

        def gbody(jj, _):
            for p in range(3):
                g = 3 * jj + 1 + p
                ib = (1 + p) % 3
                nxt = (ib + 1) % 3
                idx_fire(jnp.minimum(g + 1, NG - 1), nxt)
                idx_wait(ib)
                do_group(ib, False)
            return 0

        lax.fori_loop(0, (NG - 1) // 3, gbody, 0)

        gather_wait(2)
        scatter_fire(0, GC - 2, 2)
        gather_wait(3)
        scatter_fire(0, GC - 1, 3)
        for b in range(4):
            scatter_wait(b)
        idx_wait(1)
        plsc.subcore_barrier()

        obase = pl.multiple_of(sid * NPAD + t * STRIPE, 16)
        for k in range(STRIPE // ZR):
            pltpu.sync_copy(acc_s.at[pl.ds(stripe + ZR * k, ZR)],
                            acc_ref.at[pl.ds(obase + ZR * k, ZR)])
        plsc.subcore_barrier()


def _dinv_packed(p_ref):
    dinv = lax.rsqrt(p_ref[0] + p_ref[1] + 1.0)
    dinv_t = dinv.T
    return jnp.concatenate(
        [jnp.broadcast_to(dinv_t[:, a:a + 1], (BR, SLW)) for a in range(8)],
        axis=1)


def _pack(v):
    return jnp.concatenate([v[BR * a:BR * (a + 1), :] for a in range(8)],
                           axis=1)


def _unpack(vp):
    return jnp.concatenate([vp[:, SLW * a:SLW * (a + 1)] for a in range(8)],
                           axis=0)


def _stage_a_body(p_ref, x_ref, w1_ref, g_ref):
    dinv_p = _dinv_packed(p_ref)
    hw = jnp.dot(x_ref[...], w1_ref[...],
                 preferred_element_type=jnp.float32)
    for s in range(NSLICE):
        g_ref[s] = _pack(hw[:, SLW * s:SLW * (s + 1)]) * dinv_p


def _stage_b_body(p_ref, acc_ref, g_ref, b1p_ref, m2_ref, g2_ref):
    dinv_p = _dinv_packed(p_ref)
    hs_p = [jnp.maximum((acc_ref[s] + g_ref[s]) * dinv_p + b1p_ref[s], 0.0)
            for s in range(NSLICE)]
    for sp in range(NSLICE):
        hw2 = jnp.zeros((BR, LW), jnp.float32)
        for s in range(NSLICE):
            hw2 = hw2 + jnp.dot(hs_p[s], m2_ref[s, sp],
                                preferred_element_type=jnp.float32)
        g2_ref[sp] = hw2 * dinv_p


def _stage_c_body(p_ref, acc_ref, g_ref, b2p_ref, ml_ref, bl_ref, o_ref):
    dinv_p = _dinv_packed(p_ref)
    logits_p = jnp.zeros((BR, 8 * OUT), jnp.float32)
    for s in range(NSLICE):
        hs_p = jnp.maximum(
            (acc_ref[s] + g_ref[s]) * dinv_p + b2p_ref[s], 0.0)
        logits_p = logits_p + jnp.dot(hs_p, ml_ref[s],
                                      preferred_element_type=jnp.float32)
    logits = jnp.concatenate(
        [logits_p[:, OUT * a:OUT * (a + 1)] for a in range(8)],
        axis=0) + bl_ref[...]
    m = jnp.max(logits, axis=1, keepdims=True)
    e = jnp.exp(logits - m)
    o_ref[...] = e / jnp.sum(e, axis=1, keepdims=True)


_SC_MESH = plsc.VectorSubcoreMesh(core_axis_name="c", subcore_axis_name="s")
_SC_PARAMS = pltpu.CompilerParams(use_tc_tiling_on_sc=False)

_deg_kernel = pl.kernel(
    _deg_body,
    out_type=jax.ShapeDtypeStruct((2 * NPAD,), jnp.float32),
    mesh=_SC_MESH,
    compiler_params=_SC_PARAMS,
    scratch_types=[
        pltpu.MemorySpace.VMEM_SHARED((NPAD,), jnp.float32),
        pltpu.VMEM((2, CE), jnp.int32),
        pltpu.VMEM((CE,), jnp.float32),
        pltpu.VMEM((STRIPE,), jnp.float32),
        pltpu.SemaphoreType.DMA,
        pltpu.SemaphoreType.DMA,
    ],
)

_agg_kernel = pl.kernel(
    _agg_body,
    out_type=jax.ShapeDtypeStruct((NSLICE * NPAD, SLW), jnp.float32),
    mesh=_SC_MESH,
    compiler_params=_SC_PARAMS,
    scratch_types=[
        pltpu.MemorySpace.VMEM_SHARED((NPAD, SLW), jnp.float32),
        pltpu.VMEM((3, GC, CE), jnp.int32),
        pltpu.VMEM((3, GC, CE), jnp.int32),
        pltpu.VMEM((4, CE, SLW), jnp.float32),
        pltpu.VMEM((ZR, SLW), jnp.float32),
        pltpu.SemaphoreType.DMA,
        pltpu.SemaphoreType.DMA,
        pltpu.SemaphoreType.DMA,
        pltpu.SemaphoreType.DMA,
        pltpu.SemaphoreType.DMA,
        pltpu.SemaphoreType.DMA,
        pltpu.SemaphoreType.DMA,
        pltpu.SemaphoreType.DMA,
        pltpu.SemaphoreType.DMA,
        pltpu.SemaphoreType.DMA,
        pltpu.SemaphoreType.DMA,
    ],
)

_stage_a = pl.pallas_call(
    _stage_a_body,
    grid=(GRID,),
    in_specs=[
        pl.BlockSpec((2, 8, LW), lambda i: (0, i, 0)),
        pl.BlockSpec((R, F_IN), lambda i: (i, 0)),
        pl.BlockSpec((F_IN, H), lambda i: (0, 0)),
    ],
    out_specs=pl.BlockSpec((NSLICE, BR, LW), lambda i: (0, i, 0)),
    out_shape=jax.ShapeDtypeStruct((NSLICE, NPAD // 8, LW), jnp.float32),
)

_stage_b = pl.pallas_call(
    _stage_b_body,
    grid=(GRID,),
    in_specs=[
        pl.BlockSpec((2, 8, LW), lambda i: (0, i, 0)),
        pl.BlockSpec((NSLICE, BR, LW), lambda i: (0, i, 0)),
        pl.BlockSpec((NSLICE, BR, LW), lambda i: (0, i, 0)),
        pl.BlockSpec((NSLICE, LW), lambda i: (0, 0)),
        pl.BlockSpec((NSLICE, NSLICE, LW, LW), lambda i: (0, 0, 0, 0)),
    ],
    out_specs=pl.BlockSpec((NSLICE, BR, LW), lambda i: (0, i, 0)),
    out_shape=jax.ShapeDtypeStruct((NSLICE, NPAD // 8, LW), jnp.float32),
)

_stage_c = pl.pallas_call(
    _stage_c_body,
    grid=(GRID,),
    in_specs=[
        pl.BlockSpec((2, 8, LW), lambda i: (0, i, 0)),
        pl.BlockSpec((NSLICE, BR, LW), lambda i: (0, i, 0)),
        pl.BlockSpec((NSLICE, BR, LW), lambda i: (0, i, 0)),
        pl.BlockSpec((NSLICE, LW), lambda i: (0, 0)),
        pl.BlockSpec((NSLICE, LW, 8 * OUT), lambda i: (0, 0, 0)),
        pl.BlockSpec((1, OUT), lambda i: (0, 0)),
    ],
    out_specs=pl.BlockSpec((R, OUT), lambda i: (i, 0)),
    out_shape=jax.ShapeDtypeStruct((N, OUT), jnp.float32),
)


def _pack_bias(b):
    return jnp.broadcast_to(b.reshape(NSLICE, 1, SLW),
                            (NSLICE, 8, SLW)).reshape(NSLICE, LW)


def _block_diag_w(w):
    eye8 = jnp.eye(8, dtype=w.dtype)
    return jnp.stack([jnp.kron(eye8, w[SLW * s:SLW * (s + 1), :])
                      for s in range(NSLICE)])


def _block_diag_w2(w2):
    eye8 = jnp.eye(8, dtype=w2.dtype)
    return jnp.stack([
        jnp.stack([jnp.kron(eye8,
                            w2[SLW * s:SLW * (s + 1), SLW * sp:SLW * (sp + 1)])
                   for sp in range(NSLICE)])
        for s in range(NSLICE)])


def _perm(n):
    return (n & ~1023) | ((n & 127) << 3) | ((n >> 7) & 7)


def kernel(x, edge_index, W1, b1, W2, b2, Wl, bl):
    pad_src = jnp.arange(PAD, dtype=jnp.int32) % N
    pad_dst = N + (jnp.arange(PAD, dtype=jnp.int32) % 8)
    ep = jnp.concatenate(
        [edge_index.astype(jnp.int32), jnp.stack([pad_src, pad_dst])], axis=1)
    dstf = ep[1]
    ep2 = _perm(ep)
    src2 = ep2[0].reshape(-1, CE)
    dst2 = ep2[1].reshape(-1, CE)

    p = _deg_kernel(dstf).reshape(2, PR, LW)

    g1 = _stage_a(p, x, W1)
    acc1 = _agg_kernel(g1.reshape(NSLICE * NPAD, SLW), src2, dst2)
    g2 = _stage_b(p, acc1.reshape(NSLICE, NPAD // 8, LW), g1,
                  _pack_bias(b1), _block_diag_w2(W2))
    acc2 = _agg_kernel(g2.reshape(NSLICE * NPAD, SLW), src2, dst2)
    return _stage_c(p, acc2.reshape(NSLICE, NPAD // 8, LW), g2,
                    _pack_bias(b2), _block_diag_w(Wl), bl.reshape(1, OUT))
